# baseline (speedup 1.0000x reference)
import functools

import jax
import jax.numpy as jnp
from jax import lax
from jax.experimental import pallas as pl
from jax.experimental.pallas import tpu as pltpu
from jax.experimental.pallas import tpu_sc as plsc

_L = 16


@functools.lru_cache(maxsize=None)
def _build(B, V, D):
    info = plsc.get_sparse_core_info()
    nc, ns = info.num_cores, info.num_subcores
    nw = nc * ns
    assert B % (8 * nw) == 0
    bpw = B // nw
    groups = bpw // _L

    mesh = plsc.VectorSubcoreMesh(core_axis_name="c", subcore_axis_name="s")

    @functools.partial(
        pl.kernel,
        mesh=mesh,
        out_type=jax.ShapeDtypeStruct((B,), jnp.float32),
        compiler_params=pltpu.CompilerParams(
            needs_layout_passes=False, use_tc_tiling_on_sc=False
        ),
        scratch_types=[
            pltpu.VMEM((bpw,), jnp.int32),
            pltpu.VMEM((bpw,), jnp.int32),
            pltpu.VMEM((bpw, D), jnp.float32),
            pltpu.VMEM((bpw, D), jnp.float32),
            pltpu.VMEM((bpw,), jnp.float32),
            pltpu.VMEM((bpw,), jnp.float32),
            pltpu.VMEM((bpw,), jnp.float32),
            pltpu.VMEM((_L * _L,), jnp.float32),
            pltpu.SemaphoreType.DMA,
        ],
    )
    def glove_kernel(i_hbm, j_hbm, wi_hbm, wj_hbm, bi_hbm, bj_hbm, out_hbm,
                     ii_v, jj_v, ri_v, rj_v, bi_v, bj_v, out_v, pt_v, sem):
        wid = lax.axis_index("s") * nc + lax.axis_index("c")
        base = wid * bpw

        pltpu.sync_copy(i_hbm.at[pl.ds(base, bpw)], ii_v)
        pltpu.sync_copy(j_hbm.at[pl.ds(base, bpw)], jj_v)

        c1 = pltpu.async_copy(wi_hbm.at[ii_v], ri_v, sem)
        c2 = pltpu.async_copy(wj_hbm.at[jj_v], rj_v, sem)
        c3 = pltpu.async_copy(bi_hbm.at[ii_v], bi_v, sem)
        c4 = pltpu.async_copy(bj_hbm.at[jj_v], bj_v, sem)
        c1.wait()
        c2.wait()
        c3.wait()
        c4.wait()

        col0 = lax.iota(jnp.int32, _L) * _L

        def group(g, carry):
            row0 = pl.multiple_of(g * _L, _L)
            for k in range(_L):
                r = row0 + k
                s = None
                for c in range(D // _L):
                    p = ri_v[r, pl.ds(c * _L, _L)] * rj_v[r, pl.ds(c * _L, _L)]
                    s = p if s is None else s + p
                plsc.store_scatter(pt_v, [col0 + k], s)
            acc = bi_v[pl.ds(row0, _L)] + bj_v[pl.ds(row0, _L)]
            for l in range(_L):
                acc = acc + pt_v[pl.ds(l * _L, _L)]
            out_v[pl.ds(row0, _L)] = acc
            return carry

        lax.fori_loop(0, groups, group, 0)
        pltpu.sync_copy(out_v, out_hbm.at[pl.ds(base, bpw)])

    return glove_kernel


def kernel(i_indices, j_indices, wi, wj, bi, bj):
    V, D = wi.shape
    B = i_indices.shape[0]
    fn = _build(B, V, D)
    return fn(
        i_indices.astype(jnp.int32),
        j_indices.astype(jnp.int32),
        wi,
        wj,
        bi.reshape(V),
        bj.reshape(V),
    )
